# Initial kernel scaffold; baseline (speedup 1.0000x reference)
import jax
import jax.numpy as jnp
from jax.experimental import pallas as pl


def kernel(x, edge_index, query_embedding, W1, b1, W2, b2, Wfc, bfc):
    raise NotImplementedError("write your pallas kernel here")


# speedup vs baseline: 6.5264x; 6.5264x over previous
import functools

import jax
import jax.numpy as jnp
from jax import lax
from jax.experimental import pallas as pl
from jax.experimental.pallas import tpu as pltpu
from jax.experimental.pallas import tpu_sc as plsc

N = 10000
E = 160000
H = 256
NP = 10112
RPT = 632
EP = 163840
ROWB = 1000
GRID = N // ROWB

def _zero_rows(zsrc, acc, base, bounce):
    pltpu.sync_copy(zsrc, bounce)
    for k in range(4):
        pltpu.sync_copy(bounce, acc.at[pl.ds(base + k * 128, 128)])
    pltpu.sync_copy(bounce.at[pl.ds(0, RPT - 512)],
                    acc.at[pl.ds(base + 512, RPT - 512)])


def _dump_rows(acc, base, bounce, out, obase):
    for k in range(4):
        pltpu.sync_copy(acc.at[pl.ds(base + k * 128, 128)], bounce)
        pltpu.sync_copy(bounce, out.at[pl.ds(obase + k * 128, 128)])
    pltpu.sync_copy(acc.at[pl.ds(base + 512, RPT - 512)],
                    bounce.at[pl.ds(0, RPT - 512)])
    pltpu.sync_copy(bounce.at[pl.ds(0, RPT - 512)],
                    out.at[pl.ds(obase + 512, RPT - 512)])


@functools.cache
def _sc_degree_kernel():
    mesh = plsc.VectorSubcoreMesh(core_axis_name="c", subcore_axis_name="s")
    return functools.partial(
        pl.kernel,
        out_type=jax.ShapeDtypeStruct((2 * NP, 128), jnp.float32),
        mesh=mesh,
        scratch_types=[
            pltpu.VMEM((40, 128), jnp.int32),
            pltpu.VMEM((128, 128), jnp.float32),
            pltpu.VMEM((128, 128), jnp.float32),
            pltpu.VMEM_SHARED((NP, 128), jnp.float32),
        ],
    )(_sc_degree_body)


def _sc_degree_body(dst3, ones_hbm, z_hbm, out, idxv, onesv, bounce, acc):
    cid = lax.axis_index("c")
    sid = lax.axis_index("s")
    wid = sid * 2 + cid
    base = sid * RPT

    pltpu.sync_copy(dst3.at[wid], idxv)
    pltpu.sync_copy(ones_hbm, onesv)
    _zero_rows(z_hbm, acc, base, bounce)
    plsc.subcore_barrier()

    def body(j, carry):
        pltpu.sync_copy(onesv, acc.at[idxv.at[j]], add=True)
        return carry

    lax.fori_loop(0, 40, body, 0)
    plsc.subcore_barrier()
    _dump_rows(acc, base, bounce, out, cid * NP + base)


@functools.cache
def _sc_scatter_kernel():
    mesh = plsc.VectorSubcoreMesh(core_axis_name="c", subcore_axis_name="s")
    return functools.partial(
        pl.kernel,
        out_type=jax.ShapeDtypeStruct((2 * NP, 128), jnp.float32),
        mesh=mesh,
        scratch_types=[
            pltpu.VMEM((80, 128), jnp.int32),
            pltpu.VMEM((80, 128), jnp.int32),
            pltpu.VMEM((128, 128), jnp.float32),
            pltpu.VMEM_SHARED((NP, 128), jnp.float32),
            pltpu.SemaphoreType.DMA,
        ],
    )(_sc_scatter_body)


def _sc_scatter_body(ysL, ysR, src3, dst3, z_hbm, out, srcv, dstv, gbuf, acc, sem):
    cid = lax.axis_index("c")
    sid = lax.axis_index("s")
    base = sid * RPT

    pltpu.sync_copy(src3.at[sid], srcv)
    pltpu.sync_copy(dst3.at[sid], dstv)
    _zero_rows(z_hbm, acc, base, gbuf)
    plsc.subcore_barrier()

    def chunk(tbl, j):
        pltpu.async_copy(tbl.at[srcv.at[j]], gbuf, sem).wait()
        pltpu.sync_copy(gbuf, acc.at[dstv.at[j]], add=True)

    @pl.when(cid == 0)
    def _():
        lax.fori_loop(0, 80, lambda j, c: (chunk(ysL, j), c)[1], 0)

    @pl.when(cid == 1)
    def _():
        lax.fori_loop(0, 80, lambda j, c: (chunk(ysR, j), c)[1], 0)

    plsc.subcore_barrier()
    _dump_rows(acc, base, gbuf, out, cid * NP + base)


def _dinv(dA, dB):
    return lax.rsqrt(dA[:, 0:1] + dB[:, 0:1] + 1.0)


def _scale_body(x_ref, w_ref, dA_ref, dB_ref, outL_ref, outR_ref):
    y = jnp.dot(x_ref[...], w_ref[...], preferred_element_type=jnp.float32,
                precision=lax.Precision.HIGHEST)
    ys = y * _dinv(dA_ref[...], dB_ref[...])
    outL_ref[...] = ys[:, :128]
    outR_ref[...] = ys[:, 128:]


def _mid_body(aggL_ref, aggR_ref, ysL_ref, ysR_ref, dA_ref, dB_ref, w_ref,
              b_ref, outL_ref, outR_ref):
    dinv = _dinv(dA_ref[...], dB_ref[...])
    h = jnp.concatenate([aggL_ref[...] + ysL_ref[...],
                         aggR_ref[...] + ysR_ref[...]], axis=1)
    h = jnp.maximum(dinv * h + b_ref[...], 0.0)
    y = jnp.dot(h, w_ref[...], preferred_element_type=jnp.float32,
                precision=lax.Precision.HIGHEST)
    ys = y * dinv
    outL_ref[...] = ys[:, :128]
    outR_ref[...] = ys[:, 128:]


def _final_body(aggL_ref, aggR_ref, ysL_ref, ysR_ref, dA_ref, dB_ref, b_ref,
                wh_ref, wq_ref, q_ref, bfc_ref, out_ref):
    dinv = _dinv(dA_ref[...], dB_ref[...])
    h = jnp.concatenate([aggL_ref[...] + ysL_ref[...],
                         aggR_ref[...] + ysR_ref[...]], axis=1)
    h = jnp.maximum(dinv * h + b_ref[...], 0.0)
    const = jnp.sum(q_ref[...] * wq_ref[...]) + bfc_ref[0, 0]
    out_ref[...] = jnp.sum(h * wh_ref[...], axis=1, keepdims=True) + const


def _row_spec(w):
    return pl.BlockSpec((ROWB, w), lambda i: (i, 0))


def _full_spec(h, w):
    return pl.BlockSpec((h, w), lambda i: (0, 0))


_tc_scale = pl.pallas_call(
    _scale_body,
    grid=(GRID,),
    in_specs=[_row_spec(H), _full_spec(H, H), _row_spec(128), _row_spec(128)],
    out_specs=[_row_spec(128), _row_spec(128)],
    out_shape=[jax.ShapeDtypeStruct((N, 128), jnp.float32)] * 2,
)

_tc_mid = pl.pallas_call(
    _mid_body,
    grid=(GRID,),
    in_specs=[_row_spec(128), _row_spec(128), _row_spec(128), _row_spec(128),
              _row_spec(128), _row_spec(128), _full_spec(H, H), _full_spec(1, H)],
    out_specs=[_row_spec(128), _row_spec(128)],
    out_shape=[jax.ShapeDtypeStruct((N, 128), jnp.float32)] * 2,
)

_tc_final = pl.pallas_call(
    _final_body,
    grid=(GRID,),
    in_specs=[_row_spec(128), _row_spec(128), _row_spec(128), _row_spec(128),
              _row_spec(128), _row_spec(128), _full_spec(1, H), _full_spec(1, H),
              _full_spec(1, H), _full_spec(1, H), _full_spec(1, 1)],
    out_specs=_row_spec(1),
    out_shape=jax.ShapeDtypeStruct((N, 1), jnp.float32),
)


def kernel(x, edge_index, query_embedding, W1, b1, W2, b2, Wfc, bfc):
    src = edge_index[0].astype(jnp.int32)
    dst = edge_index[1].astype(jnp.int32)
    pad = EP - E
    srcp = jnp.concatenate([src, jnp.zeros((pad,), jnp.int32)])
    dstp = jnp.concatenate([dst, jnp.full((pad,), N, jnp.int32)])
    dst_deg = dstp.reshape(32, 40, 128)
    src_sc = srcp.reshape(16, 80, 128)
    dst_sc = dstp.reshape(16, 80, 128)
    ones128 = jnp.ones((128, 128), jnp.float32)
    zz = jnp.zeros((128, 128), jnp.float32)

    degp = _sc_degree_kernel()(dst_deg, ones128, zz)
    dA, dB = degp[:NP], degp[NP:]

    ysL, ysR = _tc_scale(x, W1, dA[:N], dB[:N])
    agg1 = _sc_scatter_kernel()(ysL, ysR, src_sc, dst_sc, zz)
    y2L, y2R = _tc_mid(agg1[:N], agg1[NP:NP + N], ysL, ysR, dA[:N], dB[:N],
                       W2, b1.reshape(1, H))
    agg2 = _sc_scatter_kernel()(y2L, y2R, src_sc, dst_sc, zz)
    scores = _tc_final(agg2[:N], agg2[NP:NP + N], y2L, y2R, dA[:N], dB[:N],
                       b2.reshape(1, H), Wfc[:H].reshape(1, H),
                       Wfc[H:].reshape(1, H), query_embedding.reshape(1, H),
                       bfc.reshape(1, 1))
    return scores[:, 0]


# speedup vs baseline: 7.5815x; 1.1617x over previous
import functools

import jax
import jax.numpy as jnp
from jax import lax
from jax.experimental import pallas as pl
from jax.experimental.pallas import tpu as pltpu
from jax.experimental.pallas import tpu_sc as plsc

N = 10000
E = 160000
H = 256
NP = 10112
RPT = 632
CH = 128
NCH = 80
NBLK = 10
NCHD = 40
EP = 163840
ROWB = 1000
GRID = N // ROWB


_NFULL = RPT // CH
_TAIL = RPT % CH


def _zero_rows(acc, base, bounce):
    for k in range(_NFULL):
        pltpu.sync_copy(bounce, acc.at[pl.ds(base + k * CH, CH)])
    pltpu.sync_copy(bounce.at[pl.ds(0, _TAIL)],
                    acc.at[pl.ds(base + _NFULL * CH, _TAIL)])


def _dump_rows(acc, base, bounce, out, obase):
    for k in range(_NFULL):
        pltpu.sync_copy(acc.at[pl.ds(base + k * CH, CH)], bounce)
        pltpu.sync_copy(bounce, out.at[pl.ds(obase + k * CH, CH)])
    pltpu.sync_copy(acc.at[pl.ds(base + _NFULL * CH, _TAIL)],
                    bounce.at[pl.ds(0, _TAIL)])
    pltpu.sync_copy(bounce.at[pl.ds(0, _TAIL)],
                    out.at[pl.ds(obase + _NFULL * CH, _TAIL)])


@functools.cache
def _sc_degree_kernel():
    mesh = plsc.VectorSubcoreMesh(core_axis_name="c", subcore_axis_name="s")
    return functools.partial(
        pl.kernel,
        out_type=jax.ShapeDtypeStruct((2 * NP, 128), jnp.float32),
        mesh=mesh,
        scratch_types=[
            pltpu.VMEM((NCHD, CH), jnp.int32),
            pltpu.VMEM((CH, 128), jnp.float32),
            pltpu.VMEM((CH, 128), jnp.float32),
            pltpu.VMEM_SHARED((NP, 128), jnp.float32),
        ],
    )(_sc_degree_body)


def _sc_degree_body(dst3, ones_hbm, z_hbm, out, idxv, onesv, bounce, acc):
    cid = lax.axis_index("c")
    sid = lax.axis_index("s")
    wid = sid * 2 + cid
    base = sid * RPT

    pltpu.sync_copy(dst3.at[wid], idxv)
    pltpu.sync_copy(ones_hbm, onesv)
    pltpu.sync_copy(z_hbm, bounce)
    _zero_rows(acc, base, bounce)
    plsc.subcore_barrier()

    def body(j, carry):
        pltpu.sync_copy(onesv, acc.at[idxv.at[j]], add=True)
        return carry

    lax.fori_loop(0, NCHD, body, 0)
    plsc.subcore_barrier()
    _dump_rows(acc, base, bounce, out, cid * NP + base)


@functools.cache
def _sc_scatter_kernel():
    mesh = plsc.VectorSubcoreMesh(core_axis_name="c", subcore_axis_name="s")
    return functools.partial(
        pl.kernel,
        out_type=jax.ShapeDtypeStruct((2 * NP, 128), jnp.float32),
        mesh=mesh,
        scratch_types=[
            pltpu.VMEM((32, 128), jnp.int32),
            pltpu.VMEM((CH, 128), jnp.float32),
            pltpu.VMEM((CH, 128), jnp.float32),
            pltpu.VMEM_SHARED((NP, 128), jnp.float32),
            pltpu.SemaphoreType.DMA,
            pltpu.SemaphoreType.DMA,
            pltpu.SemaphoreType.DMA,
        ],
    )(_sc_scatter_body)


def _sc_scatter_body(ysL, ysR, sd, z_hbm, out,
                     ring, gA, gB, acc, semA, semB, semI):
    cid = lax.axis_index("c")
    sid = lax.axis_index("s")
    base = sid * RPT
    tbase = sid * (16 * NBLK)

    pltpu.sync_copy(z_hbm, gA)
    _zero_rows(acc, base, gA)
    pltpu.async_copy(sd.at[pl.ds(tbase, 16)], ring.at[pl.ds(0, 16)], semI)
    plsc.subcore_barrier()

    def run(tbl):
        def block(b, carry):
            hh = (b % 2) * 16
            pltpu.make_async_copy(sd.at[pl.ds(tbase, 16)],
                                  ring.at[pl.ds(0, 16)], semI).wait()

            @pl.when(b < NBLK - 1)
            def _():
                nh = 16 - hh
                pltpu.async_copy(sd.at[pl.ds(tbase + (b + 1) * 16, 16)],
                                 ring.at[pl.ds(nh, 16)], semI)

            pltpu.async_copy(tbl.at[ring.at[hh]], gA, semA)
            for r in range(4):
                c0 = 2 * r
                pltpu.make_async_copy(tbl.at[ring.at[hh + c0]], gA, semA).wait()
                pltpu.async_copy(tbl.at[ring.at[hh + c0 + 1]], gB, semB)
                pltpu.sync_copy(gA, acc.at[ring.at[hh + 8 + c0]], add=True)
                if r < 3:
                    pltpu.async_copy(tbl.at[ring.at[hh + c0 + 2]], gA, semA)
                pltpu.make_async_copy(tbl.at[ring.at[hh + c0 + 1]], gB, semB).wait()
                pltpu.sync_copy(gB, acc.at[ring.at[hh + 8 + c0 + 1]], add=True)
            return carry

        lax.fori_loop(0, NBLK, block, 0)

    @pl.when(cid == 0)
    def _():
        run(ysL)

    @pl.when(cid == 1)
    def _():
        run(ysR)

    plsc.subcore_barrier()
    _dump_rows(acc, base, gA, out, cid * NP + base)


def _dinv(dA, dB):
    return lax.rsqrt(dA[:, 0:1] + dB[:, 0:1] + 1.0)


def _scale_body(x_ref, w_ref, dA_ref, dB_ref, outL_ref, outR_ref):
    y = jnp.dot(x_ref[...], w_ref[...], preferred_element_type=jnp.float32,
                precision=lax.Precision.HIGHEST)
    ys = y * _dinv(dA_ref[...], dB_ref[...])
    outL_ref[...] = ys[:, :128]
    outR_ref[...] = ys[:, 128:]


def _mid_body(aggL_ref, aggR_ref, ysL_ref, ysR_ref, dA_ref, dB_ref, w_ref,
              b_ref, outL_ref, outR_ref):
    dinv = _dinv(dA_ref[...], dB_ref[...])
    h = jnp.concatenate([aggL_ref[...] + ysL_ref[...],
                         aggR_ref[...] + ysR_ref[...]], axis=1)
    h = jnp.maximum(dinv * h + b_ref[...], 0.0)
    y = jnp.dot(h, w_ref[...], preferred_element_type=jnp.float32,
                precision=lax.Precision.HIGHEST)
    ys = y * dinv
    outL_ref[...] = ys[:, :128]
    outR_ref[...] = ys[:, 128:]


def _final_body(aggL_ref, aggR_ref, ysL_ref, ysR_ref, dA_ref, dB_ref, b_ref,
                wh_ref, wq_ref, q_ref, bfc_ref, out_ref):
    dinv = _dinv(dA_ref[...], dB_ref[...])
    h = jnp.concatenate([aggL_ref[...] + ysL_ref[...],
                         aggR_ref[...] + ysR_ref[...]], axis=1)
    h = jnp.maximum(dinv * h + b_ref[...], 0.0)
    const = jnp.sum(q_ref[...] * wq_ref[...]) + bfc_ref[0, 0]
    out_ref[...] = jnp.sum(h * wh_ref[...], axis=1, keepdims=True) + const


def _row_spec(w):
    return pl.BlockSpec((ROWB, w), lambda i: (i, 0))


def _full_spec(h, w):
    return pl.BlockSpec((h, w), lambda i: (0, 0))


_tc_scale = pl.pallas_call(
    _scale_body,
    grid=(GRID,),
    in_specs=[_row_spec(H), _full_spec(H, H), _row_spec(128), _row_spec(128)],
    out_specs=[_row_spec(128), _row_spec(128)],
    out_shape=[jax.ShapeDtypeStruct((N, 128), jnp.float32)] * 2,
)

_tc_mid = pl.pallas_call(
    _mid_body,
    grid=(GRID,),
    in_specs=[_row_spec(128), _row_spec(128), _row_spec(128), _row_spec(128),
              _row_spec(128), _row_spec(128), _full_spec(H, H), _full_spec(1, H)],
    out_specs=[_row_spec(128), _row_spec(128)],
    out_shape=[jax.ShapeDtypeStruct((N, 128), jnp.float32)] * 2,
)

_tc_final = pl.pallas_call(
    _final_body,
    grid=(GRID,),
    in_specs=[_row_spec(128), _row_spec(128), _row_spec(128), _row_spec(128),
              _row_spec(128), _row_spec(128), _full_spec(1, H), _full_spec(1, H),
              _full_spec(1, H), _full_spec(1, H), _full_spec(1, 1)],
    out_specs=_row_spec(1),
    out_shape=jax.ShapeDtypeStruct((N, 1), jnp.float32),
)


def kernel(x, edge_index, query_embedding, W1, b1, W2, b2, Wfc, bfc):
    src = edge_index[0].astype(jnp.int32)
    dst = edge_index[1].astype(jnp.int32)
    pad = EP - E
    srcp = jnp.concatenate([src, jnp.zeros((pad,), jnp.int32)])
    dstp = jnp.concatenate([dst, jnp.full((pad,), N, jnp.int32)])
    dst_deg = dstp.reshape(32, NCHD, CH)
    s4 = srcp.reshape(16, NBLK, 8, CH)
    d4 = dstp.reshape(16, NBLK, 8, CH)
    sd = jnp.concatenate([s4, d4], axis=2).reshape(16 * NBLK * 16, CH)
    ones128 = jnp.ones((CH, 128), jnp.float32)
    zz = jnp.zeros((CH, 128), jnp.float32)

    degp = _sc_degree_kernel()(dst_deg, ones128, zz)
    dA, dB = degp[:NP], degp[NP:]

    ysL, ysR = _tc_scale(x, W1, dA[:N], dB[:N])
    agg1 = _sc_scatter_kernel()(ysL, ysR, sd, zz)
    y2L, y2R = _tc_mid(agg1[:N], agg1[NP:NP + N], ysL, ysR, dA[:N], dB[:N],
                       W2, b1.reshape(1, H))
    agg2 = _sc_scatter_kernel()(y2L, y2R, sd, zz)
    scores = _tc_final(agg2[:N], agg2[NP:NP + N], y2L, y2R, dA[:N], dB[:N],
                       b2.reshape(1, H), Wfc[:H].reshape(1, H),
                       Wfc[H:].reshape(1, H), query_embedding.reshape(1, H),
                       bfc.reshape(1, 1))
    return scores[:, 0]
